# baseline (speedup 1.0000x reference)
import functools

import jax
import jax.numpy as jnp
from jax import lax
from jax.experimental import pallas as pl
from jax.experimental.pallas import tpu as pltpu
from jax.experimental.pallas import tpu_sc as plsc

B = 64
L = 576
KEPT = 144
D = 768
NB = 32
XS = KEPT + 2
MROW = XS - 1
OR_ = L + 1
RPT = 36
GN = RPT + 1
GPAD = 48

_mesh = plsc.VectorSubcoreMesh(core_axis_name="c", subcore_axis_name="s")


@functools.partial(
    pl.kernel,
    mesh=_mesh,
    out_type=jax.ShapeDtypeStruct((B * OR_, D), jnp.float32),
    scratch_types=[
        pltpu.VMEM((L,), jnp.int32),
        pltpu.VMEM((GPAD,), jnp.int32),
        pltpu.VMEM((GN, D), jnp.float32),
        pltpu.VMEM_SHARED((2 * XS, D), jnp.float32),
        pltpu.VMEM_SHARED((2 * OR_, D), jnp.float32),
        pltpu.SemaphoreType.DMA,
        pltpu.SemaphoreType.DMA,
        pltpu.SemaphoreType.DMA,
    ],
    compiler_params=pltpu.CompilerParams(
        needs_layout_passes=False, use_tc_tiling_on_sc=False
    ),
)
def _gather_kernel(x, ids, mask, out, ids_v, idx_v, buf, xs, os_, ssem, wsem, gsem):
    c = lax.axis_index("c")
    s = lax.axis_index("s")

    def stage_x(i):
        b = c * NB + i
        return pltpu.make_async_copy(
            x.at[pl.ds(b * (KEPT + 1), KEPT + 1)],
            xs.at[pl.ds((i & 1) * XS, KEPT + 1)],
            ssem,
        )

    def writeback(i):
        b = c * NB + i
        return pltpu.make_async_copy(
            os_.at[pl.ds((i & 1) * OR_, OR_)],
            out.at[pl.ds(b * OR_, OR_)],
            wsem,
        )

    @pl.when(s == 0)
    def _():
        pltpu.sync_copy(mask, xs.at[pl.ds(MROW, 1)])
        pltpu.sync_copy(mask, xs.at[pl.ds(XS + MROW, 1)])
        stage_x(0).start()

    for i in range(NB):
        slot = i & 1
        b = c * NB + i

        @pl.when(s == 0)
        def _(i=i):
            stage_x(i).wait()
        if i >= 2:
            @pl.when(s == 1)
            def _(i=i):
                writeback(i - 2).wait()
        plsc.subcore_barrier()

        if i + 1 < NB:
            @pl.when(s == 0)
            def _(i=i):
                stage_x(i + 1).start()

        pltpu.sync_copy(ids.at[pl.ds(b * L, L)], ids_v)
        for j in range(GPAD // 16):
            q = s * RPT + j * 16 + lax.iota(jnp.int32, 16)
            ii = jnp.clip(q - 1, 0, L - 1)
            pid = plsc.load_gather(ids_v, [ii])
            g = jnp.where(
                q == 0,
                0,
                jnp.where(pid < KEPT, 1 + pid, MROW),
            )
            idx_v[pl.ds(j * 16, 16)] = slot * XS + jnp.where(q <= L, g, 0)
        pltpu.async_copy(xs.at[idx_v.at[pl.ds(0, GN)]], buf, gsem).wait()
        pltpu.sync_copy(
            buf.at[pl.ds(0, RPT)],
            os_.at[pl.ds(slot * OR_ + s * RPT, RPT)],
        )
        @pl.when(s == 15)
        def _(slot=slot):
            pltpu.sync_copy(
                buf.at[pl.ds(RPT, 1)],
                os_.at[pl.ds(slot * OR_ + L, 1)],
            )
        plsc.subcore_barrier()

        @pl.when(s == 1)
        def _(i=i):
            writeback(i).start()

    @pl.when(s == 1)
    def _():
        writeback(NB - 2).wait()
        writeback(NB - 1).wait()


def kernel(x, ids_restore, mask_token):
    out = _gather_kernel(
        x.reshape(B * (KEPT + 1), D),
        ids_restore.reshape(-1).astype(jnp.int32),
        mask_token.reshape(1, D),
    )
    return out.reshape(B, OR_, D)
